# Initial kernel scaffold; baseline (speedup 1.0000x reference)
import jax
import jax.numpy as jnp
from jax.experimental import pallas as pl


def kernel(x, Wr, br, W1, b1, W2, b2):
    raise NotImplementedError("write your pallas kernel here")


# speedup vs baseline: 1.1026x; 1.1026x over previous
import functools

import jax
import jax.numpy as jnp
from jax.experimental import pallas as pl
from jax.experimental.pallas import tpu as pltpu

H = 1024
F = 4096
E = 8
S = 2048
FC = 4
FB = F // FC


def _coef_for_expert(x, wr, br, e):
    logits = jnp.dot(x, wr, preferred_element_type=jnp.float32) + br
    lmax = jnp.max(logits, axis=-1, keepdims=True)
    ex = jnp.exp(logits - lmax)
    probs = ex / jnp.sum(ex, axis=-1, keepdims=True)
    ids = jax.lax.broadcasted_iota(jnp.int32, probs.shape, 1)
    m1 = jnp.max(probs, axis=-1, keepdims=True)
    i1 = jnp.min(jnp.where(probs == m1, ids, E), axis=-1, keepdims=True)
    probs2 = jnp.where(ids == i1, -1.0, probs)
    m2 = jnp.max(probs2, axis=-1, keepdims=True)
    i2 = jnp.min(jnp.where(probs2 == m2, ids, E), axis=-1, keepdims=True)
    denom = m1 + m2
    w1 = m1 / denom
    w2 = m2 / denom
    return jnp.where(i1 == e, w1, 0.0) + jnp.where(i2 == e, w2, 0.0)


def _dense_body(x_ref, wr_ref, br_ref, w1_ref, b1_ref, w2_ref, b2_ref, out_ref):
    e = pl.program_id(0)
    f = pl.program_id(1)
    x = x_ref[...]
    coef = _coef_for_expert(x, wr_ref[...], br_ref[0], e)
    h = jnp.maximum(
        jnp.dot(x, w1_ref[0], preferred_element_type=jnp.float32) + b1_ref[0, 0], 0.0)
    y = jnp.dot(h, w2_ref[0], preferred_element_type=jnp.float32)

    @pl.when((e == 0) & (f == 0))
    def _():
        out_ref[...] = jnp.zeros_like(out_ref)

    contrib = y + jnp.where(f == 0, 1.0, 0.0) * b2_ref[0, 0]
    out_ref[...] += coef * contrib


def kernel(x, Wr, br, W1, b1, W2, b2):
    xf = x.reshape(S, H)
    br2 = br.reshape(1, E)
    b1r = b1.reshape(E, 1, F)
    b2r = b2.reshape(E, 1, H)
    out = pl.pallas_call(
        _dense_body,
        grid=(E, FC),
        in_specs=[
            pl.BlockSpec((S, H), lambda e, f: (0, 0)),
            pl.BlockSpec((H, E), lambda e, f: (0, 0)),
            pl.BlockSpec((1, E), lambda e, f: (0, 0)),
            pl.BlockSpec((1, H, FB), lambda e, f: (e, 0, f)),
            pl.BlockSpec((1, 1, FB), lambda e, f: (e, 0, f)),
            pl.BlockSpec((1, FB, H), lambda e, f: (e, f, 0)),
            pl.BlockSpec((1, 1, H), lambda e, f: (e, 0, 0)),
        ],
        out_specs=pl.BlockSpec((S, H), lambda e, f: (0, 0)),
        out_shape=jax.ShapeDtypeStruct((S, H), jnp.float32),
    )(xf, Wr, br2, W1, b1r, W2, b2r)
    return out.reshape(x.shape)


# speedup vs baseline: 1.1124x; 1.0089x over previous
import functools

import jax
import jax.numpy as jnp
from jax import lax
from jax.experimental import pallas as pl
from jax.experimental.pallas import tpu as pltpu
from jax.experimental.pallas import tpu_sc as plsc

H = 1024
F = 4096
E = 8
S = 2048
K = 2
A = S * K
BM = 128
T = A + E * BM
NBLK = T // BM
FC = 2
FB = F // FC
RCH = 256
JCH = 512


def _router_body(x_ref, wr_ref, br_ref, dest_ref, src_ref, wsrt_ref, bexp_ref):
    x = x_ref[...]
    logits = jnp.dot(x, wr_ref[...], preferred_element_type=jnp.float32)
    logits = logits + br_ref[0]
    lmax = jnp.max(logits, axis=-1, keepdims=True)
    ex = jnp.exp(logits - lmax)
    probs = ex / jnp.sum(ex, axis=-1, keepdims=True)
    ids = lax.broadcasted_iota(jnp.int32, probs.shape, 1)
    m1 = jnp.max(probs, axis=-1, keepdims=True)
    i1 = jnp.min(jnp.where(probs == m1, ids, E), axis=-1, keepdims=True)
    probs2 = jnp.where(ids == i1, -1.0, probs)
    m2 = jnp.max(probs2, axis=-1, keepdims=True)
    i2 = jnp.min(jnp.where(probs2 == m2, ids, E), axis=-1, keepdims=True)
    denom = m1 + m2
    wa = jnp.concatenate([m1 / denom, m2 / denom], axis=0)
    mask = jnp.concatenate(
        [(ids == i1).astype(jnp.float32), (ids == i2).astype(jnp.float32)],
        axis=0)

    tri = (lax.broadcasted_iota(jnp.int32, (RCH, RCH), 0)
           > lax.broadcasted_iota(jnp.int32, (RCH, RCH), 1)).astype(jnp.float32)
    carry = jnp.zeros((1, E), dtype=jnp.float32)
    chunks = []
    for c in range(A // RCH):
        blk = mask[c * RCH:(c + 1) * RCH, :]
        chunks.append(jnp.dot(tri, blk, preferred_element_type=jnp.float32)
                      + carry)
        carry = carry + jnp.sum(blk, axis=0, keepdims=True)
    ranks = jnp.concatenate(chunks, axis=0)

    counts = carry
    cpad = jnp.floor((counts + (BM - 1)) / BM) * BM
    up = (lax.broadcasted_iota(jnp.int32, (E, E), 0)
          < lax.broadcasted_iota(jnp.int32, (E, E), 1)).astype(jnp.float32)
    offs = jnp.dot(cpad, up, preferred_element_type=jnp.float32)

    dest = jnp.sum(mask * (offs + ranks), axis=-1, keepdims=True)
    dest_ref[...] = dest.astype(jnp.int32)

    tvals = (lax.broadcasted_iota(jnp.int32, (A, 1), 0) % S).astype(jnp.float32)
    for c in range(T // JCH):
        jrow = (lax.broadcasted_iota(jnp.int32, (1, JCH), 1) + c * JCH
                ).astype(jnp.float32)
        eq = (dest == jrow).astype(jnp.float32)
        src_c = jnp.sum(eq * tvals, axis=0, keepdims=True)
        w_c = jnp.sum(eq * wa, axis=0, keepdims=True)
        src_ref[c, :] = src_c.astype(jnp.int32)[0]
        wsrt_ref[c, :] = w_c[0]

    bstart = (lax.broadcasted_iota(jnp.int32, (NBLK, 1), 0) * BM
              ).astype(jnp.float32)
    inb = jnp.logical_and(bstart >= offs, bstart < offs + cpad)
    evals = lax.broadcasted_iota(jnp.int32, (NBLK, E), 1).astype(jnp.float32)
    bexp = jnp.sum(jnp.where(inb, evals, 0.0), axis=-1, keepdims=True)
    bexp_ref[...] = bexp.astype(jnp.int32)


def _router(xf, Wr, br):
    return pl.pallas_call(
        _router_body,
        grid=(1,),
        in_specs=[
            pl.BlockSpec((S, H), lambda i: (0, 0)),
            pl.BlockSpec((H, E), lambda i: (0, 0)),
            pl.BlockSpec((1, E), lambda i: (0, 0)),
        ],
        out_specs=[
            pl.BlockSpec((A, 1), lambda i: (0, 0)),
            pl.BlockSpec((T // JCH, JCH), lambda i: (0, 0)),
            pl.BlockSpec((T // JCH, JCH), lambda i: (0, 0)),
            pl.BlockSpec((NBLK, 1), lambda i: (0, 0)),
        ],
        out_shape=[
            jax.ShapeDtypeStruct((A, 1), jnp.int32),
            jax.ShapeDtypeStruct((T // JCH, JCH), jnp.int32),
            jax.ShapeDtypeStruct((T // JCH, JCH), jnp.float32),
            jax.ShapeDtypeStruct((NBLK, 1), jnp.int32),
        ],
    )(xf, Wr, br.reshape(1, E))


def _make_sc_gather(n_rows, n_cols):
    info = plsc.get_sparse_core_info()
    nw = info.num_cores * info.num_subcores
    rows_per_w = n_rows // nw
    ch = 32
    n_ch = rows_per_w // ch
    mesh = plsc.VectorSubcoreMesh(core_axis_name="c", subcore_axis_name="s")

    @functools.partial(
        pl.kernel, mesh=mesh,
        out_type=jax.ShapeDtypeStruct((n_rows, n_cols), jnp.float32),
        scratch_types=[
            pltpu.VMEM((ch,), jnp.int32),
            pltpu.VMEM((ch, n_cols), jnp.float32),
            pltpu.SemaphoreType.DMA,
        ],
    )
    def gather_k(table_hbm, idx_hbm, out_hbm, idx_v, rows_v, sem):
        wid = lax.axis_index("s") * info.num_cores + lax.axis_index("c")
        base = wid * rows_per_w

        def body(c, carry):
            off = base + c * ch
            pltpu.sync_copy(idx_hbm.at[pl.ds(off, ch)], idx_v)
            pltpu.async_copy(table_hbm.at[idx_v], rows_v, sem).wait()
            pltpu.sync_copy(rows_v, out_hbm.at[pl.ds(off, ch)])
            return carry

        lax.fori_loop(0, n_ch, body, 0)

    return gather_k


def _gmm_body(bexp_ref, xs_ref, w1_ref, b1_ref, w2_ref, b2_ref, wsrt_ref,
              ys_ref):
    f = pl.program_id(0)
    m = pl.program_id(1)
    xs = xs_ref[...]
    h = jnp.maximum(
        jnp.dot(xs, w1_ref[0], preferred_element_type=jnp.float32)
        + b1_ref[0, 0], 0.0)
    part = jnp.dot(h, w2_ref[0], preferred_element_type=jnp.float32)
    row = pl.ds(m * BM, BM)

    @pl.when(f == 0)
    def _():
        ys_ref[row, :] = part

    @pl.when(f == FC - 1)
    def _():
        ys_ref[row, :] = (ys_ref[row, :] + part + b2_ref[0, 0]) * wsrt_ref[...]


def _grouped_mlp(xs, W1, b1, W2, b2, wsrt, bexp):
    grid_spec = pltpu.PrefetchScalarGridSpec(
        num_scalar_prefetch=1,
        grid=(FC, NBLK),
        in_specs=[
            pl.BlockSpec((BM, H), lambda f, m, be: (m, 0)),
            pl.BlockSpec((1, H, FB), lambda f, m, be: (be[m], 0, f)),
            pl.BlockSpec((1, 1, FB), lambda f, m, be: (be[m], 0, f)),
            pl.BlockSpec((1, FB, H), lambda f, m, be: (be[m], f, 0)),
            pl.BlockSpec((1, 1, H), lambda f, m, be: (be[m], 0, 0)),
            pl.BlockSpec((BM, 1), lambda f, m, be: (m, 0)),
        ],
        out_specs=pl.BlockSpec((T, H), lambda f, m, be: (0, 0)),
    )
    return pl.pallas_call(
        _gmm_body,
        grid_spec=grid_spec,
        out_shape=jax.ShapeDtypeStruct((T, H), jnp.float32),
    )(bexp, xs, W1, b1.reshape(E, 1, F), W2, b2.reshape(E, 1, H), wsrt)


def _combine_body(y0_ref, y1_ref, out_ref):
    out_ref[...] = y0_ref[...] + y1_ref[...]


def _combine(y0, y1):
    return pl.pallas_call(
        _combine_body,
        grid=(4,),
        in_specs=[
            pl.BlockSpec((S // 4, H), lambda i: (i, 0)),
            pl.BlockSpec((S // 4, H), lambda i: (i, 0)),
        ],
        out_specs=pl.BlockSpec((S // 4, H), lambda i: (i, 0)),
        out_shape=jax.ShapeDtypeStruct((S, H), jnp.float32),
    )(y0, y1)


def kernel(x, Wr, br, W1, b1, W2, b2):
    xf = x.reshape(S, H)
    dest, src, wsrt, bexp = _router(xf, Wr, br)
    src_flat = src.reshape(T)
    dest_flat = dest.reshape(A)
    xs = _make_sc_gather(T, H)(xf, src_flat)
    ys = _grouped_mlp(xs, W1, b1, W2, b2, wsrt.reshape(T, 1), bexp.reshape(NBLK))
    yg = _make_sc_gather(A, H)(ys, dest_flat)
    out = _combine(yg[:S], yg[S:])
    return out.reshape(x.shape)


# speedup vs baseline: 1.1245x; 1.0109x over previous
import functools

import jax
import jax.numpy as jnp
from jax import lax
from jax.experimental import pallas as pl
from jax.experimental.pallas import tpu as pltpu
from jax.experimental.pallas import tpu_sc as plsc

H = 1024
F = 4096
E = 8
S = 2048
K = 2
A = S * K
BM = 128
T = A + E * BM
NBLK = T // BM
FC = 2
FB = F // FC
RCH = 256
JCH = 512


def _router_body(x_ref, wr_ref, br_ref, dest_ref, srcw_ref, bexp_ref):
    x = x_ref[...]
    logits = jnp.dot(x, wr_ref[...], preferred_element_type=jnp.float32)
    logits = logits + br_ref[0]
    lmax = jnp.max(logits, axis=-1, keepdims=True)
    ex = jnp.exp(logits - lmax)
    probs = ex / jnp.sum(ex, axis=-1, keepdims=True)
    ids = lax.broadcasted_iota(jnp.int32, probs.shape, 1)
    m1 = jnp.max(probs, axis=-1, keepdims=True)
    i1 = jnp.min(jnp.where(probs == m1, ids, E), axis=-1, keepdims=True)
    probs2 = jnp.where(ids == i1, -1.0, probs)
    m2 = jnp.max(probs2, axis=-1, keepdims=True)
    i2 = jnp.min(jnp.where(probs2 == m2, ids, E), axis=-1, keepdims=True)
    denom = m1 + m2
    wa = jnp.concatenate([m1 / denom, m2 / denom], axis=0)
    mask = jnp.concatenate(
        [(ids == i1).astype(jnp.float32), (ids == i2).astype(jnp.float32)],
        axis=0)

    tri = (lax.broadcasted_iota(jnp.int32, (RCH, RCH), 0)
           > lax.broadcasted_iota(jnp.int32, (RCH, RCH), 1)).astype(jnp.float32)
    carry = jnp.zeros((1, E), dtype=jnp.float32)
    chunks = []
    for c in range(A // RCH):
        blk = mask[c * RCH:(c + 1) * RCH, :]
        chunks.append(jnp.dot(tri, blk, preferred_element_type=jnp.float32)
                      + carry)
        carry = carry + jnp.sum(blk, axis=0, keepdims=True)
    ranks = jnp.concatenate(chunks, axis=0)

    counts = carry
    cpad = jnp.floor((counts + (BM - 1)) / BM) * BM
    up = (lax.broadcasted_iota(jnp.int32, (E, E), 0)
          < lax.broadcasted_iota(jnp.int32, (E, E), 1)).astype(jnp.float32)
    offs = jnp.dot(cpad, up, preferred_element_type=jnp.float32)

    dest = jnp.sum(mask * (offs + ranks), axis=-1, keepdims=True)
    dest_ref[...] = dest.astype(jnp.int32)

    tvals = (lax.broadcasted_iota(jnp.int32, (A, 1), 0) % S).astype(jnp.float32)
    vals2 = jnp.concatenate([tvals, wa], axis=1)
    for c in range(T // JCH):
        jrow = (lax.broadcasted_iota(jnp.int32, (1, JCH), 1) + c * JCH
                ).astype(jnp.float32)
        eq = (dest == jrow).astype(jnp.float32)
        res = lax.dot_general(eq, vals2, (((0,), (0,)), ((), ())),
                              preferred_element_type=jnp.float32)
        srcw_ref[pl.ds(c * JCH, JCH), :] = res

    bstart = (lax.broadcasted_iota(jnp.int32, (NBLK, 1), 0) * BM
              ).astype(jnp.float32)
    inb = jnp.logical_and(bstart >= offs, bstart < offs + cpad)
    evals = lax.broadcasted_iota(jnp.int32, (NBLK, E), 1).astype(jnp.float32)
    bexp = jnp.sum(jnp.where(inb, evals, 0.0), axis=-1, keepdims=True)
    bexp_ref[...] = bexp.astype(jnp.int32)


def _router(xf, Wr, br):
    return pl.pallas_call(
        _router_body,
        grid=(1,),
        in_specs=[
            pl.BlockSpec((S, H), lambda i: (0, 0)),
            pl.BlockSpec((H, E), lambda i: (0, 0)),
            pl.BlockSpec((1, E), lambda i: (0, 0)),
        ],
        out_specs=[
            pl.BlockSpec((A, 1), lambda i: (0, 0)),
            pl.BlockSpec((T, 2), lambda i: (0, 0)),
            pl.BlockSpec((NBLK, 1), lambda i: (0, 0)),
        ],
        out_shape=[
            jax.ShapeDtypeStruct((A, 1), jnp.int32),
            jax.ShapeDtypeStruct((T, 2), jnp.float32),
            jax.ShapeDtypeStruct((NBLK, 1), jnp.int32),
        ],
    )(xf, Wr, br.reshape(1, E))


def _make_sc_gather(n_rows, n_cols):
    info = plsc.get_sparse_core_info()
    nw = info.num_cores * info.num_subcores
    rows_per_w = n_rows // nw
    ch = 32
    n_ch = rows_per_w // ch
    mesh = plsc.VectorSubcoreMesh(core_axis_name="c", subcore_axis_name="s")

    @functools.partial(
        pl.kernel, mesh=mesh,
        out_type=jax.ShapeDtypeStruct((n_rows, n_cols), jnp.float32),
        scratch_types=[
            pltpu.VMEM((rows_per_w,), jnp.int32),
            pltpu.VMEM((ch, n_cols), jnp.float32),
            pltpu.VMEM((ch, n_cols), jnp.float32),
            pltpu.SemaphoreType.DMA,
            pltpu.SemaphoreType.DMA,
            pltpu.SemaphoreType.DMA,
            pltpu.SemaphoreType.DMA,
        ],
    )
    def gather_k(table_hbm, idx_hbm, out_hbm, idx_v, buf0, buf1, g0, g1, s0, s1):
        wid = lax.axis_index("s") * info.num_cores + lax.axis_index("c")
        base = wid * rows_per_w
        pltpu.sync_copy(idx_hbm.at[pl.ds(base, rows_per_w)], idx_v)
        bufs = (buf0, buf1)
        gsem = (g0, g1)
        ssem = (s0, s1)

        gathers = [None] * n_ch
        stores = [None] * n_ch
        gathers[0] = pltpu.async_copy(
            table_hbm.at[idx_v.at[pl.ds(0, ch)]], bufs[0], gsem[0])
        for c in range(n_ch):
            p = c & 1
            if c + 1 < n_ch:
                q = (c + 1) & 1
                if c >= 1:
                    stores[c - 1].wait()
                gathers[c + 1] = pltpu.async_copy(
                    table_hbm.at[idx_v.at[pl.ds((c + 1) * ch, ch)]],
                    bufs[q], gsem[q])
            gathers[c].wait()
            stores[c] = pltpu.async_copy(
                bufs[p], out_hbm.at[pl.ds(base + c * ch, ch)], ssem[p])
        if n_ch >= 2:
            stores[n_ch - 2].wait()
        stores[n_ch - 1].wait()

    return gather_k


def _gmm_body(bexp_ref, xs_ref, w1_ref, b1_ref, w2_ref, b2_ref, wsrt_ref,
              ys_ref):
    f = pl.program_id(0)
    m = pl.program_id(1)
    xs = xs_ref[...]
    h = jnp.maximum(
        jnp.dot(xs, w1_ref[0], preferred_element_type=jnp.float32)
        + b1_ref[0, 0], 0.0)
    part = jnp.dot(h, w2_ref[0], preferred_element_type=jnp.float32)
    row = pl.ds(m * BM, BM)

    @pl.when(f == 0)
    def _():
        ys_ref[row, :] = part

    @pl.when(f == FC - 1)
    def _():
        ys_ref[row, :] = (ys_ref[row, :] + part + b2_ref[0, 0]) * wsrt_ref[...]


def _grouped_mlp(xs, W1, b1, W2, b2, wsrt, bexp):
    grid_spec = pltpu.PrefetchScalarGridSpec(
        num_scalar_prefetch=1,
        grid=(FC, NBLK),
        in_specs=[
            pl.BlockSpec((BM, H), lambda f, m, be: (m, 0)),
            pl.BlockSpec((1, H, FB), lambda f, m, be: (be[m], 0, f)),
            pl.BlockSpec((1, 1, FB), lambda f, m, be: (be[m], 0, f)),
            pl.BlockSpec((1, FB, H), lambda f, m, be: (be[m], f, 0)),
            pl.BlockSpec((1, 1, H), lambda f, m, be: (be[m], 0, 0)),
            pl.BlockSpec((BM, 1), lambda f, m, be: (m, 0)),
        ],
        out_specs=pl.BlockSpec((T, H), lambda f, m, be: (0, 0)),
    )
    return pl.pallas_call(
        _gmm_body,
        grid_spec=grid_spec,
        out_shape=jax.ShapeDtypeStruct((T, H), jnp.float32),
    )(bexp, xs, W1, b1.reshape(E, 1, F), W2, b2.reshape(E, 1, H), wsrt)


def _combine_body(y0_ref, y1_ref, out_ref):
    out_ref[...] = y0_ref[...] + y1_ref[...]


def _combine(y0, y1):
    return pl.pallas_call(
        _combine_body,
        grid=(4,),
        in_specs=[
            pl.BlockSpec((S // 4, H), lambda i: (i, 0)),
            pl.BlockSpec((S // 4, H), lambda i: (i, 0)),
        ],
        out_specs=pl.BlockSpec((S // 4, H), lambda i: (i, 0)),
        out_shape=jax.ShapeDtypeStruct((S, H), jnp.float32),
    )(y0, y1)


def kernel(x, Wr, br, W1, b1, W2, b2):
    xf = x.reshape(S, H)
    dest, srcw, bexp = _router(xf, Wr, br)
    src_flat = srcw[:, 0].astype(jnp.int32)
    wsrt = srcw[:, 1].reshape(T, 1)
    dest_flat = dest.reshape(A)
    xs = _make_sc_gather(T, H)(xf, src_flat)
    ys = _grouped_mlp(xs, W1, b1, W2, b2, wsrt, bexp.reshape(NBLK))
    yg = _make_sc_gather(A, H)(ys, dest_flat)
    out = _combine(yg[:S], yg[S:])
    return out.reshape(x.shape)
